# baseline (speedup 1.0000x reference)
import jax
import jax.numpy as jnp
from jax import lax
from jax.experimental import pallas as pl
from jax.experimental.pallas import tpu as pltpu
from jax.experimental.pallas import tpu_sc as plsc

T = 50000
D = 128
M = 200000
NT = 100000

NC = 2
NS = 16
NW = NC * NS
L = 16

CHUNK = 128
ITEMS_PER_W = 6400
MP = NW * ITEMS_PER_W
NCHUNKS = ITEMS_PER_W // CHUNK
NSUPER = NCHUNKS // 2
NG = D // L

CP = 51200
AP = 102400
CCHUNK = 6400


def _fast_sqrt(ss):
    ib = lax.bitcast_convert_type(ss, jnp.int32)
    y = lax.bitcast_convert_type(jnp.int32(0x5F3759DF) - (ib >> 1),
                                 jnp.float32)
    for _ in range(3):
        y = y * (1.5 - (0.5 * ss) * y * y)
    return ss * y


def _body(x_hbm, a_hbm, c_hbm, ri_hbm, cj_hbm, pk_hbm, tm_hbm, wt_hbm,
          mk_hbm, ic_hbm, out_hbm,
          ri0, ri1, pk0, pk1, tm0, tm1,
          cj0, cj1, wt0, wt1, ic0, ic1,
          ck0, ck1, ci0, ci1, av0, av1,
          xk0, xk1, xi0, xi1, mk0, mk1,
          cd_v, dl_v, w2_v, cbuf_v, stage_v,
          sem_gi, sem_gc, sem_g):
    RI, PK, TM = (ri0, ri1), (pk0, pk1), (tm0, tm1)
    CJ, WT, IC = (cj0, cj1), (wt0, wt1), (ic0, ic1)
    CK, CI, AV = (ck0, ck1), (ci0, ci1), (av0, av1)
    XK, XI, MK = (xk0, xk1), (xi0, xi1), (mk0, mk1)

    wid = lax.axis_index("s") * NC + lax.axis_index("c")
    wbase = pl.multiple_of(wid * ITEMS_PER_W, ITEMS_PER_W)
    lanes = lax.iota(jnp.int32, L)

    def sl_of(g):
        return pl.ds(pl.multiple_of(wbase + g * CHUNK, CHUNK), CHUNK)

    def gi_copies(g, b):
        sl = sl_of(g)
        return [(pk_hbm.at[sl], PK[b]), (ri_hbm.at[sl], RI[b]),
                (tm_hbm.at[sl], TM[b])]

    def gc_copies(g, b):
        sl = sl_of(g)
        return [(cj_hbm.at[sl], CJ[b]), (wt_hbm.at[sl], WT[b]),
                (ic_hbm.at[sl], IC[b])]

    def g_copies(g, b):
        return [(x_hbm.at[PK[b]], XK[b]), (x_hbm.at[RI[b]], XI[b]),
                (c_hbm.at[PK[b]], CK[b]), (c_hbm.at[RI[b]], CI[b]),
                (a_hbm.at[TM[b]], AV[b]),
                (mk_hbm.at[pl.ds(pl.multiple_of(
                    (wbase + g * CHUNK) // 4, CHUNK // 4), CHUNK // 4)],
                 MK[b])]

    def issue(copies, sem):
        for src, dst in copies:
            pltpu.async_copy(src, dst, sem)

    def drain(copies, sem):
        for src, dst in copies:
            pltpu.make_async_copy(src, dst, sem).wait()

    def compute_chunk(b, acc):
        for u in range(CHUNK // L):
            s = pl.ds(u * L, L)
            ci = CI[b][s]
            cd_v[s] = CK[b][s] - ci
            dl_v[s] = AV[b][s] - ci
            w2_v[s] = WT[b][s] * IC[b][s]

        xk_v, xi_v, mk_v, cj_v = XK[b], XI[b], MK[b], CJ[b]

        def group_body(u, acc):
            gsl = pl.ds(u * L, L)
            cdg = cd_v[gsl]
            dlg = dl_v[gsl]
            jg = cj_v[gsl]
            w2g = w2_v[gsl]
            ss16 = jnp.zeros((L,), jnp.float32)
            for q in range(L):
                m = u * L + q
                cdb = jnp.full((L,), cdg[q])
                dlb = jnp.full((L,), dlg[q])
                jb = jg[q]
                mrow = u * 4 + q // 4
                mcol = (q % 4) * 32
                mw = (mk_v[mrow, pl.ds(mcol, L)],
                      mk_v[mrow, pl.ds(mcol + L, L)])
                acc16 = jnp.zeros((L,), jnp.float32)
                for c in range(NG):
                    s = pl.ds(c * L, L)
                    mbit = mw[c // 4] & jnp.int32(1 << (8 * (c % 4)))
                    t = xk_v[m, s] - xi_v[m, s] + cdb
                    t = t - jnp.where(lanes + (c * L) == jb, dlb,
                                      jnp.float32(0.0))
                    tmsk = jnp.where(mbit != 0, t, jnp.float32(0.0))
                    acc16 = acc16 + tmsk * t
                ss16 = jnp.where(lanes == q, jnp.sum(acc16), ss16)
            return acc + w2g * _fast_sqrt(ss16)

        return lax.fori_loop(0, CHUNK // L, group_body, acc)

    issue(gi_copies(0, 0), sem_gi)
    issue(gc_copies(0, 0), sem_gc)
    drain(gi_copies(0, 0), sem_gi)
    issue(g_copies(0, 0), sem_g)
    issue(gi_copies(1, 1), sem_gi)
    issue(gc_copies(1, 1), sem_gc)

    def super_body(it, acc):
        for b in (0, 1):
            g = 2 * it + b
            drain(g_copies(g, b), sem_g)

            def advance():
                drain(gi_copies(g + 1, 1 - b), sem_gi)
                issue(g_copies(g + 1, 1 - b), sem_g)

            if b == 0:
                advance()
            else:
                pl.when(it < NSUPER - 1)(advance)
            pl.when(it < NSUPER - 1)(
                lambda: issue(gi_copies(g + 2, b), sem_gi))

            drain(gc_copies(g, b), sem_gc)
            acc = compute_chunk(b, acc)
            pl.when(it < NSUPER - 1)(
                lambda: issue(gc_copies(g + 2, b), sem_gc))
        return acc

    acc = lax.fori_loop(0, NSUPER, super_body,
                        jnp.zeros((L,), jnp.float32))

    def table_norm(tab_hbm, nchunks, csize, acc):
        def tchunk(h, sq):
            pltpu.sync_copy(tab_hbm.at[pl.ds(h * csize, csize)], cbuf_v)

            def tstep(u, sq):
                v = cbuf_v[pl.ds(u * L, L)]
                return sq + v * v

            return lax.fori_loop(0, csize // L, tstep, sq)

        sq = lax.fori_loop(0, nchunks, tchunk,
                           jnp.zeros((L,), jnp.float32))
        tot = jnp.sum(sq)
        nrm = _fast_sqrt(jnp.full((L,), tot, jnp.float32))
        return acc + nrm * (1.0 / L)

    acc = lax.cond(wid == 0,
                   lambda a: table_norm(c_hbm, CP // CCHUNK, CCHUNK, a),
                   lambda a: a, acc)
    acc = lax.cond(wid == 1,
                   lambda a: table_norm(a_hbm, AP // CCHUNK, CCHUNK, a),
                   lambda a: a, acc)

    stage_v[:] = acc
    pltpu.sync_copy(stage_v, out_hbm.at[pl.ds(wid * L, L)])


@jax.jit
def kernel(X, A, C, rows_i, cols_j, peers_k, tmap, weights, masks, inv_cnt):
    padm = MP - M
    ri = jnp.pad(rows_i.astype(jnp.int32), (0, padm))
    cj = jnp.pad(cols_j.astype(jnp.int32), (0, padm))
    pk = jnp.pad(peers_k.astype(jnp.int32), (0, padm))
    tm = jnp.pad(tmap.astype(jnp.int32), (0, padm))
    wt = jnp.pad(weights, (0, padm))
    ic = jnp.pad(inv_cnt, (0, padm))
    mku8 = masks.astype(jnp.uint8).reshape(M, 2, 4, L).transpose(0, 1, 3, 2)
    mk = jnp.pad(
        lax.bitcast_convert_type(mku8, jnp.int32).reshape(M * 2 * L),
        (0, padm * 2 * L)).reshape(MP // 4, D)
    ap = jnp.pad(A, (0, AP - NT))
    cp = jnp.pad(C, (0, CP - T))

    mesh = plsc.VectorSubcoreMesh(core_axis_name="c", subcore_axis_name="s",
                                  num_cores=NC, num_subcores=NS)
    run = pl.kernel(
        _body,
        out_type=jax.ShapeDtypeStruct((NW * L,), jnp.float32),
        mesh=mesh,
        compiler_params=pltpu.CompilerParams(needs_layout_passes=False),
        scratch_types=(
            [pltpu.VMEM((CHUNK,), jnp.int32)] * 6
            + [pltpu.VMEM((CHUNK,), jnp.int32)] * 2
            + [pltpu.VMEM((CHUNK,), jnp.float32)] * 4
            + [pltpu.VMEM((CHUNK,), jnp.float32)] * 6
            + [pltpu.VMEM((CHUNK, D), jnp.float32)] * 4
            + [pltpu.VMEM((CHUNK // 4, D), jnp.int32)] * 2
            + [pltpu.VMEM((CHUNK,), jnp.float32)] * 3
            + [pltpu.VMEM((CCHUNK,), jnp.float32)]
            + [pltpu.VMEM((L,), jnp.float32)]
            + [pltpu.SemaphoreType.DMA] * 3
        ),
    )
    partials = run(X, ap, cp, ri, cj, pk, tm, wt, mk, ic)
    return jnp.sum(partials)


# speedup vs baseline: 2.0107x; 2.0107x over previous
import jax
import jax.numpy as jnp
from jax import lax
from jax.experimental import pallas as pl
from jax.experimental.pallas import tpu as pltpu
from jax.experimental.pallas import tpu_sc as plsc

T = 50000
D = 128
M = 200000
NT = 100000

NC = 2
NS = 16
NW = NC * NS
L = 16

CHUNK = 128
ITEMS_PER_W = 6400
MP = NW * ITEMS_PER_W
NCHUNKS = ITEMS_PER_W // CHUNK
NSUPER = NCHUNKS // 2
NG = D // L

CP = 51200
AP = 102400
CCHUNK = 6400


def _fast_sqrt(ss):
    ib = lax.bitcast_convert_type(ss, jnp.int32)
    y = lax.bitcast_convert_type(jnp.int32(0x5F3759DF) - (ib >> 1),
                                 jnp.float32)
    for _ in range(3):
        y = y * (1.5 - (0.5 * ss) * y * y)
    return ss * y


def _body(x_hbm, a_hbm, c_hbm, ri_hbm, cj_hbm, pk_hbm, tm_hbm, wt_hbm,
          mk_hbm, ic_hbm, out_hbm,
          ri0, ri1, pk0, pk1, tm0, tm1,
          cj0, cj1, wt0, wt1, ic0, ic1,
          ck0, ck1, ci0, ci1, av0, av1,
          xk0, xk1, xi0, xi1, mk0, mk1,
          cd_v, dl_v, w2_v, cbuf_v, stage_v,
          sem_gi, sem_gc, sem_g):
    RI, PK, TM = (ri0, ri1), (pk0, pk1), (tm0, tm1)
    CJ, WT, IC = (cj0, cj1), (wt0, wt1), (ic0, ic1)
    CK, CI, AV = (ck0, ck1), (ci0, ci1), (av0, av1)
    XK, XI, MK = (xk0, xk1), (xi0, xi1), (mk0, mk1)

    wid = lax.axis_index("s") * NC + lax.axis_index("c")
    wbase = pl.multiple_of(wid * ITEMS_PER_W, ITEMS_PER_W)
    lanes = lax.iota(jnp.int32, L)

    def sl_of(g):
        return pl.ds(pl.multiple_of(wbase + g * CHUNK, CHUNK), CHUNK)

    def gi_copies(g, b):
        sl = sl_of(g)
        return [(pk_hbm.at[sl], PK[b]), (ri_hbm.at[sl], RI[b]),
                (tm_hbm.at[sl], TM[b])]

    def gc_copies(g, b):
        sl = sl_of(g)
        return [(cj_hbm.at[sl], CJ[b]), (wt_hbm.at[sl], WT[b]),
                (ic_hbm.at[sl], IC[b])]

    def g_copies(g, b):
        return [(x_hbm.at[PK[b]], XK[b]), (x_hbm.at[RI[b]], XI[b]),
                (c_hbm.at[PK[b]], CK[b]), (c_hbm.at[RI[b]], CI[b]),
                (a_hbm.at[TM[b]], AV[b]), (mk_hbm.at[sl_of(g)], MK[b])]

    def issue(copies, sem):
        for src, dst in copies:
            pltpu.async_copy(src, dst, sem)

    def drain(copies, sem):
        for src, dst in copies:
            pltpu.make_async_copy(src, dst, sem).wait()

    def compute_chunk(b, acc):
        for u in range(CHUNK // L):
            s = pl.ds(u * L, L)
            ci = CI[b][s]
            cd_v[s] = CK[b][s] - ci
            dl_v[s] = AV[b][s] - ci
            w2_v[s] = WT[b][s] * IC[b][s]

        xk_v, xi_v, mk_v, cj_v = XK[b], XI[b], MK[b], CJ[b]

        def group_body(u, acc):
            gsl = pl.ds(u * L, L)
            cdg = cd_v[gsl]
            dlg = dl_v[gsl]
            jg = cj_v[gsl]
            w2g = w2_v[gsl]
            ss16 = jnp.zeros((L,), jnp.float32)
            for q in range(L):
                m = u * L + q
                cdb = jnp.full((L,), cdg[q])
                dlb = jnp.full((L,), dlg[q])
                jb = jg[q]
                acc16 = jnp.zeros((L,), jnp.float32)
                for c in range(NG):
                    s = pl.ds(c * L, L)
                    t = xk_v[m, s] - xi_v[m, s] + cdb
                    t = t - jnp.where(lanes + (c * L) == jb, dlb,
                                      jnp.float32(0.0))
                    tmsk = t * mk_v[m, s]
                    acc16 = acc16 + tmsk * t
                ss16 = jnp.where(lanes == q, jnp.sum(acc16), ss16)
            return acc + w2g * _fast_sqrt(ss16)

        return lax.fori_loop(0, CHUNK // L, group_body, acc)

    issue(gi_copies(0, 0), sem_gi)
    issue(gc_copies(0, 0), sem_gc)
    drain(gi_copies(0, 0), sem_gi)
    issue(g_copies(0, 0), sem_g)
    issue(gi_copies(1, 1), sem_gi)
    issue(gc_copies(1, 1), sem_gc)

    def super_body(it, acc):
        for b in (0, 1):
            g = 2 * it + b
            drain(g_copies(g, b), sem_g)

            def advance():
                drain(gi_copies(g + 1, 1 - b), sem_gi)
                issue(g_copies(g + 1, 1 - b), sem_g)

            if b == 0:
                advance()
            else:
                pl.when(it < NSUPER - 1)(advance)
            pl.when(it < NSUPER - 1)(
                lambda: issue(gi_copies(g + 2, b), sem_gi))

            drain(gc_copies(g, b), sem_gc)
            acc = compute_chunk(b, acc)
            pl.when(it < NSUPER - 1)(
                lambda: issue(gc_copies(g + 2, b), sem_gc))
        return acc

    acc = lax.fori_loop(0, NSUPER, super_body,
                        jnp.zeros((L,), jnp.float32))

    def table_norm(tab_hbm, nchunks, csize, acc):
        def tchunk(h, sq):
            pltpu.sync_copy(tab_hbm.at[pl.ds(h * csize, csize)], cbuf_v)

            def tstep(u, sq):
                v = cbuf_v[pl.ds(u * L, L)]
                return sq + v * v

            return lax.fori_loop(0, csize // L, tstep, sq)

        sq = lax.fori_loop(0, nchunks, tchunk,
                           jnp.zeros((L,), jnp.float32))
        tot = jnp.sum(sq)
        nrm = _fast_sqrt(jnp.full((L,), tot, jnp.float32))
        return acc + nrm * (1.0 / L)

    acc = lax.cond(wid == 0,
                   lambda a: table_norm(c_hbm, CP // CCHUNK, CCHUNK, a),
                   lambda a: a, acc)
    acc = lax.cond(wid == 1,
                   lambda a: table_norm(a_hbm, AP // CCHUNK, CCHUNK, a),
                   lambda a: a, acc)

    stage_v[:] = acc
    pltpu.sync_copy(stage_v, out_hbm.at[pl.ds(wid * L, L)])


@jax.jit
def kernel(X, A, C, rows_i, cols_j, peers_k, tmap, weights, masks, inv_cnt):
    padm = MP - M
    ri = jnp.pad(rows_i.astype(jnp.int32), (0, padm))
    cj = jnp.pad(cols_j.astype(jnp.int32), (0, padm))
    pk = jnp.pad(peers_k.astype(jnp.int32), (0, padm))
    tm = jnp.pad(tmap.astype(jnp.int32), (0, padm))
    wt = jnp.pad(weights, (0, padm))
    ic = jnp.pad(inv_cnt, (0, padm))
    mk = jnp.pad(masks.astype(jnp.float32), ((0, padm), (0, 0)))
    ap = jnp.pad(A, (0, AP - NT))
    cp = jnp.pad(C, (0, CP - T))

    mesh = plsc.VectorSubcoreMesh(core_axis_name="c", subcore_axis_name="s",
                                  num_cores=NC, num_subcores=NS)
    run = pl.kernel(
        _body,
        out_type=jax.ShapeDtypeStruct((NW * L,), jnp.float32),
        mesh=mesh,
        compiler_params=pltpu.CompilerParams(needs_layout_passes=False),
        scratch_types=(
            [pltpu.VMEM((CHUNK,), jnp.int32)] * 6
            + [pltpu.VMEM((CHUNK,), jnp.int32)] * 2
            + [pltpu.VMEM((CHUNK,), jnp.float32)] * 4
            + [pltpu.VMEM((CHUNK,), jnp.float32)] * 6
            + [pltpu.VMEM((CHUNK, D), jnp.float32)] * 6
            + [pltpu.VMEM((CHUNK,), jnp.float32)] * 3
            + [pltpu.VMEM((CCHUNK,), jnp.float32)]
            + [pltpu.VMEM((L,), jnp.float32)]
            + [pltpu.SemaphoreType.DMA] * 3
        ),
    )
    partials = run(X, ap, cp, ri, cj, pk, tm, wt, mk, ic)
    return jnp.sum(partials)
